# baseline (speedup 1.0000x reference)
import functools

import jax
import jax.numpy as jnp
from jax import lax
from jax.experimental import pallas as pl
from jax.experimental.pallas import tpu as pltpu
from jax.experimental.pallas import tpu_sc as plsc

NW = 32
LANES = 16
C_EDGES = 2000
K_GATHER = 32


def _mm_xs_body(xb, pb, wx, wv, v_out, xsb_out):
    v = jnp.dot(pb[...], wv[...], preferred_element_type=jnp.float32)
    v_out[...] = v
    xs = jnp.dot(xb[...], wx[...], preferred_element_type=jnp.float32) + v
    xsb_out[...] = xs.astype(jnp.bfloat16)


def _mm_out_body(mb, vb, blb, wg, bgb, ob):
    h = jnp.maximum(mb[...].astype(jnp.float32) - vb[...] + blb[...], 0.0)
    ob[...] = jnp.dot(h, wg[...], preferred_element_type=jnp.float32) + bgb[...]


G_RING = 8


def _sc_segmax(np_rows, dw, ep, r):
    n_rounds = ep // C_EDGES
    n_grp = C_EDGES // LANES
    n_sub = dw // LANES
    cl = C_EDGES + 3 * LANES
    mesh = plsc.VectorSubcoreMesh(core_axis_name="c", subcore_axis_name="s")

    @functools.partial(
        pl.kernel,
        mesh=mesh,
        compiler_params=pltpu.CompilerParams(needs_layout_passes=False),
        out_type=jax.ShapeDtypeStruct((np_rows, dw), jnp.int32),
        scratch_types=[
            pltpu.VMEM((r + 1, dw), jnp.int32),
            pltpu.VMEM((2 * C_EDGES,), jnp.int32),
            pltpu.VMEM((2 * C_EDGES,), jnp.int32),
            pltpu.VMEM((2 * cl,), jnp.int32),
            pltpu.VMEM((2 * cl,), jnp.int32),
            pltpu.VMEM((G_RING * K_GATHER, dw), jnp.int32),
            pltpu.SemaphoreType.DMA,
            pltpu.SemaphoreType.DMA,
        ],
    )
    def seg_max(xs_hbm, dst_hbm, src_hbm, m_hbm,
                m_loc, dbuf, sbuf, slist, dloc, rows, esem, gsem):
        cid = lax.axis_index("c")
        sid = lax.axis_index("s")
        wid = sid * 2 + cid
        lo = wid * r
        iota = lax.iota(jnp.int32, LANES)

        def start_edges(ci):
            p = ci % 2
            base = ci * C_EDGES
            pltpu.async_copy(dst_hbm.at[pl.ds(base, C_EDGES)],
                             dbuf.at[pl.ds(p * C_EDGES, C_EDGES)], esem)
            pltpu.async_copy(src_hbm.at[pl.ds(base, C_EDGES)],
                             sbuf.at[pl.ds(p * C_EDGES, C_EDGES)], esem)

        def wait_edges(p):
            pltpu.make_async_copy(
                dst_hbm.at[pl.ds(0, C_EDGES)],
                dbuf.at[pl.ds(p * C_EDGES, C_EDGES)], esem).wait()
            pltpu.make_async_copy(
                src_hbm.at[pl.ds(0, C_EDGES)],
                sbuf.at[pl.ds(p * C_EDGES, C_EDGES)], esem).wait()

        def issue(par, blk_loc, slot):
            pltpu.async_copy(
                xs_hbm.at[slist.at[pl.ds(par * cl + blk_loc * K_GATHER,
                                         K_GATHER)]],
                rows.at[pl.ds(slot * K_GATHER, K_GATHER)], gsem)

        def wait_gather(slot):
            pltpu.make_async_copy(
                xs_hbm.at[pl.ds(0, K_GATHER)],
                rows.at[pl.ds(slot * K_GATHER, K_GATHER)], gsem).wait()

        start_edges(0)
        pltpu.sync_copy(xs_hbm.at[pl.ds(lo, r)], m_loc.at[pl.ds(0, r)])

        def round_body(ci, carry):
            pq, iq, b_prev, nb_prev = carry
            p = ci % 2
            wait_edges(p)

            @pl.when(ci + 1 < n_rounds)
            def _():
                start_edges(ci + 1)

            def grp(g, n):
                d16 = dbuf[pl.ds(p * C_EDGES + g * LANES, LANES)]
                s16 = sbuf[pl.ds(p * C_EDGES + g * LANES, LANES)]
                msk = (d16 - lo).astype(jnp.uint32) < jnp.uint32(r)
                plsc.store_compressed(
                    slist.at[pl.ds(p * cl + n, LANES)], s16, mask=msk)
                plsc.store_compressed(
                    dloc.at[pl.ds(p * cl + n, LANES)], d16 - lo, mask=msk)
                return n + plsc.all_reduce_population_count(msk)[0]

            n = lax.fori_loop(0, n_grp, grp, jnp.int32(0))
            slist[pl.ds(p * cl + n, LANES)] = jnp.zeros((LANES,), jnp.int32)
            slist[pl.ds(p * cl + n + LANES, LANES)] = jnp.zeros(
                (LANES,), jnp.int32)
            dloc[pl.ds(p * cl + n, LANES)] = jnp.full((LANES,), r, jnp.int32)
            dloc[pl.ds(p * cl + n + LANES, LANES)] = jnp.full(
                (LANES,), r, jnp.int32)
            nb_cur = (n + K_GATHER - 1) // K_GATHER
            b_cur = b_prev + nb_prev
            frontier = b_cur + nb_cur

            def issue_global(q):
                is_cur = q >= b_cur
                par = jnp.where(is_cur, p, 1 - p)
                loc = q - jnp.where(is_cur, b_cur, b_prev)
                issue(par, loc, q % G_RING)

            def seed(k, iq2):
                can = (iq2 < frontier) & (iq2 - pq < G_RING)

                @pl.when(can)
                def _():
                    issue_global(iq2)

                return iq2 + can.astype(jnp.int32)

            iq = lax.fori_loop(0, G_RING, seed, iq)

            def pbody(b, carry2):
                pq2, iq2 = carry2
                g = b_prev + b

                @pl.when(iq2 == g)
                def _():
                    issue_global(g)

                iq2 = jnp.maximum(iq2, g + 1)
                slot = g % G_RING
                wait_gather(slot)

                def grp16(h, carry3):
                    dl16 = dloc[pl.ds((1 - p) * cl + b * K_GATHER + h * LANES,
                                      LANES)]

                    def edge(i, carry4):
                        dlv = dl16.at[jnp.full((LANES,), i, jnp.int32)].get(
                            mode="promise_in_bounds")
                        gs, rs = [], []
                        for j in range(n_sub):
                            gs.append(plsc.load_gather(
                                m_loc, [dlv, iota + (j * LANES)]))
                            rs.append(rows[slot * K_GATHER + h * LANES + i,
                                           pl.ds(j * LANES, LANES)])
                        mx = [
                            plsc.bitcast(
                                jnp.maximum(plsc.bitcast(a, jnp.bfloat16),
                                            plsc.bitcast(c, jnp.bfloat16)),
                                jnp.int32)
                            for a, c in zip(gs, rs)
                        ]
                        for j in range(n_sub):
                            plsc.store_scatter(
                                m_loc, [dlv, iota + (j * LANES)], mx[j])
                        return carry4

                    lax.fori_loop(0, LANES, edge, jnp.int32(0))
                    return carry3

                lax.fori_loop(0, K_GATHER // LANES, grp16, jnp.int32(0))
                pq2 = g + 1
                can = (iq2 < frontier) & (iq2 - pq2 < G_RING)

                @pl.when(can)
                def _():
                    issue_global(iq2)

                return pq2, iq2 + can.astype(jnp.int32)

            pq, iq = lax.fori_loop(0, nb_prev, pbody, (pq, iq))
            return pq, iq, b_cur, nb_cur

        z = jnp.int32(0)
        lax.fori_loop(0, n_rounds, round_body, (z, z, z, z))
        pltpu.sync_copy(m_loc.at[pl.ds(0, r)], m_hbm.at[pl.ds(lo, r)])

    return seg_max


def kernel(x, pos, edge_index, W_local, b_local, W_global, b_global):
    n, d = x.shape
    e = edge_index.shape[1]
    r = -(-n // (NW * 8)) * 8
    np_rows = NW * r
    ep = (-(-e // C_EDGES) + 1) * C_EDGES

    x_pad = jnp.zeros((np_rows, d), jnp.float32).at[:n].set(x)
    pos_pad = jnp.zeros((np_rows, 128), jnp.float32).at[:n, :3].set(pos)
    wx = W_local[:d]
    wv = jnp.zeros((128, d), jnp.float32).at[:3].set(W_local[d:])
    src = jnp.zeros((ep,), jnp.int32).at[:e].set(edge_index[0])
    dst = jnp.full((ep,), jnp.int32(1 << 30)).at[:e].set(edge_index[1])

    blk_rows = 512
    grid = (np_rows // blk_rows,)
    v, xs_bf = pl.pallas_call(
        _mm_xs_body,
        grid=grid,
        in_specs=[
            pl.BlockSpec((blk_rows, d), lambda i: (i, 0)),
            pl.BlockSpec((blk_rows, 128), lambda i: (i, 0)),
            pl.BlockSpec((d, d), lambda i: (0, 0)),
            pl.BlockSpec((128, d), lambda i: (0, 0)),
        ],
        out_specs=[
            pl.BlockSpec((blk_rows, d), lambda i: (i, 0)),
            pl.BlockSpec((blk_rows, d), lambda i: (i, 0)),
        ],
        out_shape=[
            jax.ShapeDtypeStruct((np_rows, d), jnp.float32),
            jax.ShapeDtypeStruct((np_rows, d), jnp.bfloat16),
        ],
    )(x_pad, pos_pad, wx, wv)

    dw = d // 2
    xs_pack = lax.bitcast_convert_type(
        xs_bf.reshape(np_rows, dw, 2), jnp.int32)

    m = _sc_segmax(np_rows, dw, ep, r)(xs_pack, dst, src)

    m_bf = lax.bitcast_convert_type(m, jnp.bfloat16).reshape(np_rows, d)

    out = pl.pallas_call(
        _mm_out_body,
        grid=grid,
        in_specs=[
            pl.BlockSpec((blk_rows, d), lambda i: (i, 0)),
            pl.BlockSpec((blk_rows, d), lambda i: (i, 0)),
            pl.BlockSpec((1, d), lambda i: (0, 0)),
            pl.BlockSpec((d, d), lambda i: (0, 0)),
            pl.BlockSpec((1, d), lambda i: (0, 0)),
        ],
        out_specs=pl.BlockSpec((blk_rows, d), lambda i: (i, 0)),
        out_shape=jax.ShapeDtypeStruct((np_rows, d), jnp.float32),
    )(m_bf, v, b_local.reshape(1, d), W_global, b_global.reshape(1, d))

    return out[:n]


# speedup vs baseline: 2.6306x; 2.6306x over previous
import functools

import jax
import jax.numpy as jnp
from jax import lax
from jax.experimental import pallas as pl
from jax.experimental.pallas import tpu as pltpu
from jax.experimental.pallas import tpu_sc as plsc

NW = 32
LANES = 16
C_EDGES = 2000
K_GATHER = 32


def _mm_xs_body(xb, pb, wx, wv, v_out, xsb_out):
    v = jnp.dot(pb[...], wv[...], preferred_element_type=jnp.float32)
    v_out[...] = v
    xs = jnp.dot(xb[...], wx[...], preferred_element_type=jnp.float32) + v
    xsb_out[...] = xs.astype(jnp.bfloat16)


def _mm_out_body(mb, vb, blb, wg, bgb, ob):
    h = jnp.maximum(mb[...].astype(jnp.float32) - vb[...] + blb[...], 0.0)
    ob[...] = jnp.dot(h, wg[...], preferred_element_type=jnp.float32) + bgb[...]


G_RING = 8


def _sc_segmax(np_rows, dw, ep, r, part_rows, n_parts, xsp_rows):
    n_rounds = ep // C_EDGES
    n_grp = C_EDGES // LANES
    n_sub = dw // LANES
    cl = C_EDGES + 3 * LANES
    mesh = plsc.VectorSubcoreMesh(core_axis_name="c", subcore_axis_name="s")

    @functools.partial(
        pl.kernel,
        mesh=mesh,
        compiler_params=pltpu.CompilerParams(needs_layout_passes=False),
        out_type=jax.ShapeDtypeStruct((np_rows, dw), jnp.int32),
        scratch_types=[
            pltpu.VMEM((r + 1, dw), jnp.int32),
            pltpu.VMEM((2 * C_EDGES,), jnp.int32),
            pltpu.VMEM((2 * C_EDGES,), jnp.int32),
            pltpu.VMEM((2 * cl,), jnp.int32),
            pltpu.VMEM((2 * cl,), jnp.int32),
            pltpu.VMEM((G_RING * K_GATHER, dw), jnp.int32),
            pltpu.VMEM_SHARED((part_rows, dw), jnp.int32),
            pltpu.SemaphoreType.DMA,
            pltpu.SemaphoreType.DMA,
            pltpu.SemaphoreType.DMA,
        ],
    )
    def seg_max(xs_hbm, dst_hbm, src_hbm, m_hbm,
                m_loc, dbuf, sbuf, slist, dloc, rows, xs_sp, esem, gsem, ssem):
        cid = lax.axis_index("c")
        sid = lax.axis_index("s")
        wid = sid * 2 + cid
        lo = wid * r
        iota = lax.iota(jnp.int32, LANES)

        def start_edges(ci):
            p = ci % 2
            base = ci * C_EDGES
            pltpu.async_copy(dst_hbm.at[pl.ds(base, C_EDGES)],
                             dbuf.at[pl.ds(p * C_EDGES, C_EDGES)], esem)
            pltpu.async_copy(src_hbm.at[pl.ds(base, C_EDGES)],
                             sbuf.at[pl.ds(p * C_EDGES, C_EDGES)], esem)

        def wait_edges(p):
            pltpu.make_async_copy(
                dst_hbm.at[pl.ds(0, C_EDGES)],
                dbuf.at[pl.ds(p * C_EDGES, C_EDGES)], esem).wait()
            pltpu.make_async_copy(
                src_hbm.at[pl.ds(0, C_EDGES)],
                sbuf.at[pl.ds(p * C_EDGES, C_EDGES)], esem).wait()

        def issue(par, blk_loc, slot):
            pltpu.async_copy(
                xs_sp.at[slist.at[pl.ds(par * cl + blk_loc * K_GATHER,
                                        K_GATHER)]],
                rows.at[pl.ds(slot * K_GATHER, K_GATHER)], gsem)

        def wait_gather(slot):
            pltpu.make_async_copy(
                xs_hbm.at[pl.ds(0, K_GATHER)],
                rows.at[pl.ds(slot * K_GATHER, K_GATHER)], gsem).wait()

        pltpu.sync_copy(xs_hbm.at[pl.ds(lo, r)], m_loc.at[pl.ds(0, r)])

        def round_body(base_s, ci, carry):
            pq, iq, b_prev, nb_prev = carry
            p = ci % 2
            wait_edges(p)

            @pl.when(ci + 1 < n_rounds)
            def _():
                start_edges(ci + 1)

            def grp(g, n):
                d16 = dbuf[pl.ds(p * C_EDGES + g * LANES, LANES)]
                s16 = sbuf[pl.ds(p * C_EDGES + g * LANES, LANES)]
                sl16 = s16 - base_s
                msk = ((d16 - lo).astype(jnp.uint32) < jnp.uint32(r)) & (
                    sl16.astype(jnp.uint32) < jnp.uint32(part_rows))
                plsc.store_compressed(
                    slist.at[pl.ds(p * cl + n, LANES)], sl16, mask=msk)
                plsc.store_compressed(
                    dloc.at[pl.ds(p * cl + n, LANES)], d16 - lo, mask=msk)
                return n + plsc.all_reduce_population_count(msk)[0]

            n = lax.fori_loop(0, n_grp, grp, jnp.int32(0))
            slist[pl.ds(p * cl + n, LANES)] = jnp.zeros((LANES,), jnp.int32)
            slist[pl.ds(p * cl + n + LANES, LANES)] = jnp.zeros(
                (LANES,), jnp.int32)
            dloc[pl.ds(p * cl + n, LANES)] = jnp.full((LANES,), r, jnp.int32)
            dloc[pl.ds(p * cl + n + LANES, LANES)] = jnp.full(
                (LANES,), r, jnp.int32)
            nb_cur = (n + K_GATHER - 1) // K_GATHER
            b_cur = b_prev + nb_prev
            frontier = b_cur + nb_cur

            def issue_global(q):
                is_cur = q >= b_cur
                par = jnp.where(is_cur, p, 1 - p)
                loc = q - jnp.where(is_cur, b_cur, b_prev)
                issue(par, loc, q % G_RING)

            def seed(k, iq2):
                can = (iq2 < frontier) & (iq2 - pq < G_RING)

                @pl.when(can)
                def _():
                    issue_global(iq2)

                return iq2 + can.astype(jnp.int32)

            iq = lax.fori_loop(0, G_RING, seed, iq)

            def pbody(b, carry2):
                pq2, iq2 = carry2
                g = b_prev + b

                @pl.when(iq2 == g)
                def _():
                    issue_global(g)

                iq2 = jnp.maximum(iq2, g + 1)
                slot = g % G_RING
                wait_gather(slot)

                def grp16(h, carry3):
                    dl16 = dloc[pl.ds((1 - p) * cl + b * K_GATHER + h * LANES,
                                      LANES)]

                    def edge(i, carry4):
                        dlv = dl16.at[jnp.full((LANES,), i, jnp.int32)].get(
                            mode="promise_in_bounds")
                        gs, rs = [], []
                        for j in range(n_sub):
                            gs.append(plsc.load_gather(
                                m_loc, [dlv, iota + (j * LANES)]))
                            rs.append(rows[slot * K_GATHER + h * LANES + i,
                                           pl.ds(j * LANES, LANES)])
                        mx = [
                            plsc.bitcast(
                                jnp.maximum(plsc.bitcast(a, jnp.bfloat16),
                                            plsc.bitcast(c, jnp.bfloat16)),
                                jnp.int32)
                            for a, c in zip(gs, rs)
                        ]
                        for j in range(n_sub):
                            plsc.store_scatter(
                                m_loc, [dlv, iota + (j * LANES)], mx[j])
                        return carry4

                    lax.fori_loop(0, LANES, edge, jnp.int32(0))
                    return carry3

                lax.fori_loop(0, K_GATHER // LANES, grp16, jnp.int32(0))
                pq2 = g + 1
                can = (iq2 < frontier) & (iq2 - pq2 < G_RING)

                @pl.when(can)
                def _():
                    issue_global(iq2)

                return pq2, iq2 + can.astype(jnp.int32)

            pq, iq = lax.fori_loop(0, nb_prev, pbody, (pq, iq))
            return pq, iq, b_cur, nb_cur

        z = jnp.int32(0)
        s_rows = part_rows // 16
        for t in range(n_parts):
            pltpu.async_copy(
                xs_hbm.at[pl.ds(t * part_rows + sid * s_rows, s_rows)],
                xs_sp.at[pl.ds(sid * s_rows, s_rows)], ssem)
            pltpu.make_async_copy(
                xs_hbm.at[pl.ds(0, s_rows)],
                xs_sp.at[pl.ds(sid * s_rows, s_rows)], ssem).wait()
            plsc.subcore_barrier()
            start_edges(0)
            lax.fori_loop(0, n_rounds,
                          functools.partial(round_body, t * part_rows),
                          (z, z, z, z))
            plsc.subcore_barrier()
        pltpu.sync_copy(m_loc.at[pl.ds(0, r)], m_hbm.at[pl.ds(lo, r)])

    return seg_max


def kernel(x, pos, edge_index, W_local, b_local, W_global, b_global):
    n, d = x.shape
    e = edge_index.shape[1]
    r = -(-n // (NW * 8)) * 8
    np_rows = NW * r
    ep = (-(-e // C_EDGES) + 1) * C_EDGES

    x_pad = jnp.pad(x, ((0, np_rows - n), (0, 0)))
    pos_pad = jnp.pad(pos, ((0, np_rows - n), (0, 128 - pos.shape[1])))
    wx = W_local[:d]
    wv = jnp.pad(W_local[d:], ((0, 128 - (W_local.shape[0] - d)), (0, 0)))
    src = jnp.pad(edge_index[0], (0, ep - e))
    dst = jnp.pad(edge_index[1], (0, ep - e), constant_values=1 << 30)

    blk_rows = 512
    grid = (np_rows // blk_rows,)
    v, xs_bf = pl.pallas_call(
        _mm_xs_body,
        grid=grid,
        in_specs=[
            pl.BlockSpec((blk_rows, d), lambda i: (i, 0)),
            pl.BlockSpec((blk_rows, 128), lambda i: (i, 0)),
            pl.BlockSpec((d, d), lambda i: (0, 0)),
            pl.BlockSpec((128, d), lambda i: (0, 0)),
        ],
        out_specs=[
            pl.BlockSpec((blk_rows, d), lambda i: (i, 0)),
            pl.BlockSpec((blk_rows, d), lambda i: (i, 0)),
        ],
        out_shape=[
            jax.ShapeDtypeStruct((np_rows, d), jnp.float32),
            jax.ShapeDtypeStruct((np_rows, d), jnp.bfloat16),
        ],
    )(x_pad, pos_pad, wx, wv)

    dw = d // 2
    xs_pack = lax.bitcast_convert_type(
        xs_bf.reshape(np_rows, dw, 2), jnp.int32)

    part_max = 442368 // dw // 128 * 128
    n_parts = -(-np_rows // part_max)
    part_rows = -(-np_rows // (n_parts * 128)) * 128
    xsp_rows = n_parts * part_rows
    xs_pack = jnp.pad(xs_pack, ((0, xsp_rows - np_rows), (0, 0)))

    m = _sc_segmax(np_rows, dw, ep, r, part_rows, n_parts, xsp_rows)(
        xs_pack, dst, src)

    m_bf = lax.bitcast_convert_type(m, jnp.bfloat16).reshape(np_rows, d)

    out = pl.pallas_call(
        _mm_out_body,
        grid=grid,
        in_specs=[
            pl.BlockSpec((blk_rows, d), lambda i: (i, 0)),
            pl.BlockSpec((blk_rows, d), lambda i: (i, 0)),
            pl.BlockSpec((1, d), lambda i: (0, 0)),
            pl.BlockSpec((d, d), lambda i: (0, 0)),
            pl.BlockSpec((1, d), lambda i: (0, 0)),
        ],
        out_specs=pl.BlockSpec((blk_rows, d), lambda i: (i, 0)),
        out_shape=jax.ShapeDtypeStruct((np_rows, d), jnp.float32),
    )(m_bf, v, b_local.reshape(1, d), W_global, b_global.reshape(1, d))

    return out[:n]


# speedup vs baseline: 2.7139x; 1.0316x over previous
import functools

import jax
import jax.numpy as jnp
from jax import lax
from jax.experimental import pallas as pl
from jax.experimental.pallas import tpu as pltpu
from jax.experimental.pallas import tpu_sc as plsc

NW = 32
LANES = 16
C_EDGES = 2000
K_GATHER = 32


def _mm_xs_body(xb, pb, wx, wv, v_out, xsb_out):
    v = jnp.dot(pb[...], wv[...], preferred_element_type=jnp.float32)
    v_out[...] = v
    xs = jnp.dot(xb[...].astype(jnp.bfloat16), wx[...].astype(jnp.bfloat16),
                 preferred_element_type=jnp.float32) + v
    xsb_out[...] = xs.astype(jnp.bfloat16)


def _mm_out_body(mb, vb, blb, wg, bgb, ob):
    h = jnp.maximum(mb[...].astype(jnp.float32) - vb[...] + blb[...], 0.0)
    ob[...] = jnp.dot(h.astype(jnp.bfloat16), wg[...].astype(jnp.bfloat16),
                      preferred_element_type=jnp.float32) + bgb[...]


G_RING = 8


def _sc_segmax(np_rows, dw, ep, r, part_rows, n_parts, xsp_rows):
    n_rounds = ep // C_EDGES
    n_grp = C_EDGES // LANES
    n_sub = dw // LANES
    cl = C_EDGES + 3 * LANES
    mesh = plsc.VectorSubcoreMesh(core_axis_name="c", subcore_axis_name="s")

    @functools.partial(
        pl.kernel,
        mesh=mesh,
        compiler_params=pltpu.CompilerParams(needs_layout_passes=False),
        out_type=jax.ShapeDtypeStruct((np_rows, dw), jnp.int32),
        scratch_types=[
            pltpu.VMEM((r + 1, dw), jnp.int32),
            pltpu.VMEM((2 * C_EDGES,), jnp.int32),
            pltpu.VMEM((2 * C_EDGES,), jnp.int32),
            pltpu.VMEM((2 * cl,), jnp.int32),
            pltpu.VMEM((2 * cl,), jnp.int32),
            pltpu.VMEM((G_RING * K_GATHER, dw), jnp.int32),
            pltpu.VMEM_SHARED((part_rows, dw), jnp.int32),
            pltpu.SemaphoreType.DMA,
            pltpu.SemaphoreType.DMA,
            pltpu.SemaphoreType.DMA,
        ],
    )
    def seg_max(xs_hbm, dst_hbm, src_hbm, m_hbm,
                m_loc, dbuf, sbuf, slist, dloc, rows, xs_sp, esem, gsem, ssem):
        cid = lax.axis_index("c")
        sid = lax.axis_index("s")
        wid = sid * 2 + cid
        lo = wid * r
        iota = lax.iota(jnp.int32, LANES)

        def start_edges(ci):
            p = ci % 2
            base = ci * C_EDGES
            pltpu.async_copy(dst_hbm.at[pl.ds(base, C_EDGES)],
                             dbuf.at[pl.ds(p * C_EDGES, C_EDGES)], esem)
            pltpu.async_copy(src_hbm.at[pl.ds(base, C_EDGES)],
                             sbuf.at[pl.ds(p * C_EDGES, C_EDGES)], esem)

        def wait_edges(p):
            pltpu.make_async_copy(
                dst_hbm.at[pl.ds(0, C_EDGES)],
                dbuf.at[pl.ds(p * C_EDGES, C_EDGES)], esem).wait()
            pltpu.make_async_copy(
                src_hbm.at[pl.ds(0, C_EDGES)],
                sbuf.at[pl.ds(p * C_EDGES, C_EDGES)], esem).wait()

        def issue(par, blk_loc, slot):
            pltpu.async_copy(
                xs_sp.at[slist.at[pl.ds(par * cl + blk_loc * K_GATHER,
                                        K_GATHER)]],
                rows.at[pl.ds(slot * K_GATHER, K_GATHER)], gsem)

        def wait_gather(slot):
            pltpu.make_async_copy(
                xs_hbm.at[pl.ds(0, K_GATHER)],
                rows.at[pl.ds(slot * K_GATHER, K_GATHER)], gsem).wait()

        pltpu.sync_copy(xs_hbm.at[pl.ds(lo, r)], m_loc.at[pl.ds(0, r)])

        def round_body(base_s, ci, carry):
            pq, iq, b_prev, nb_prev = carry
            p = ci % 2
            wait_edges(p)

            @pl.when(ci + 1 < n_rounds)
            def _():
                start_edges(ci + 1)

            def grp(g, n):
                d16 = dbuf[pl.ds(p * C_EDGES + g * LANES, LANES)]
                s16 = sbuf[pl.ds(p * C_EDGES + g * LANES, LANES)]
                sl16 = s16 - base_s
                msk = ((d16 - lo).astype(jnp.uint32) < jnp.uint32(r)) & (
                    sl16.astype(jnp.uint32) < jnp.uint32(part_rows))
                plsc.store_compressed(
                    slist.at[pl.ds(p * cl + n, LANES)], sl16, mask=msk)
                plsc.store_compressed(
                    dloc.at[pl.ds(p * cl + n, LANES)], d16 - lo, mask=msk)
                return n + plsc.all_reduce_population_count(msk)[0]

            n = lax.fori_loop(0, n_grp, grp, jnp.int32(0), unroll=2)
            slist[pl.ds(p * cl + n, LANES)] = jnp.zeros((LANES,), jnp.int32)
            slist[pl.ds(p * cl + n + LANES, LANES)] = jnp.zeros(
                (LANES,), jnp.int32)
            dloc[pl.ds(p * cl + n, LANES)] = jnp.full((LANES,), r, jnp.int32)
            dloc[pl.ds(p * cl + n + LANES, LANES)] = jnp.full(
                (LANES,), r, jnp.int32)
            nb_cur = (n + K_GATHER - 1) // K_GATHER
            b_cur = b_prev + nb_prev
            frontier = b_cur + nb_cur

            def issue_global(q):
                is_cur = q >= b_cur
                par = jnp.where(is_cur, p, 1 - p)
                loc = q - jnp.where(is_cur, b_cur, b_prev)
                issue(par, loc, q % G_RING)

            def seed(k, iq2):
                can = (iq2 < frontier) & (iq2 - pq < G_RING)

                @pl.when(can)
                def _():
                    issue_global(iq2)

                return iq2 + can.astype(jnp.int32)

            iq = lax.fori_loop(0, G_RING, seed, iq)

            def pbody(b, carry2):
                pq2, iq2 = carry2
                g = b_prev + b

                @pl.when(iq2 == g)
                def _():
                    issue_global(g)

                iq2 = jnp.maximum(iq2, g + 1)
                slot = g % G_RING
                wait_gather(slot)

                def grp16(h, carry3):
                    dl16 = dloc[pl.ds((1 - p) * cl + b * K_GATHER + h * LANES,
                                      LANES)]

                    def edge(i, carry4):
                        dlv = dl16.at[jnp.full((LANES,), i, jnp.int32)].get(
                            mode="promise_in_bounds")
                        gs, rs = [], []
                        for j in range(n_sub):
                            gs.append(plsc.load_gather(
                                m_loc, [dlv, iota + (j * LANES)]))
                            rs.append(rows[slot * K_GATHER + h * LANES + i,
                                           pl.ds(j * LANES, LANES)])
                        mx = [
                            plsc.bitcast(
                                jnp.maximum(plsc.bitcast(a, jnp.bfloat16),
                                            plsc.bitcast(c, jnp.bfloat16)),
                                jnp.int32)
                            for a, c in zip(gs, rs)
                        ]
                        for j in range(n_sub):
                            plsc.store_scatter(
                                m_loc, [dlv, iota + (j * LANES)], mx[j])
                        return carry4

                    lax.fori_loop(0, LANES, edge, jnp.int32(0), unroll=2)
                    return carry3

                lax.fori_loop(0, K_GATHER // LANES, grp16, jnp.int32(0))
                pq2 = g + 1
                can = (iq2 < frontier) & (iq2 - pq2 < G_RING)

                @pl.when(can)
                def _():
                    issue_global(iq2)

                return pq2, iq2 + can.astype(jnp.int32)

            pq, iq = lax.fori_loop(0, nb_prev, pbody, (pq, iq))
            return pq, iq, b_cur, nb_cur

        z = jnp.int32(0)
        s_rows = part_rows // 16
        for t in range(n_parts):
            pltpu.async_copy(
                xs_hbm.at[pl.ds(t * part_rows + sid * s_rows, s_rows)],
                xs_sp.at[pl.ds(sid * s_rows, s_rows)], ssem)
            pltpu.make_async_copy(
                xs_hbm.at[pl.ds(0, s_rows)],
                xs_sp.at[pl.ds(sid * s_rows, s_rows)], ssem).wait()
            plsc.subcore_barrier()
            start_edges(0)
            lax.fori_loop(0, n_rounds,
                          functools.partial(round_body, t * part_rows),
                          (z, z, z, z))
            plsc.subcore_barrier()
        pltpu.sync_copy(m_loc.at[pl.ds(0, r)], m_hbm.at[pl.ds(lo, r)])

    return seg_max


def kernel(x, pos, edge_index, W_local, b_local, W_global, b_global):
    n, d = x.shape
    e = edge_index.shape[1]
    r = -(-n // (NW * 8)) * 8
    np_rows = NW * r
    ep = (-(-e // C_EDGES) + 1) * C_EDGES

    x_pad = jnp.pad(x, ((0, np_rows - n), (0, 0)))
    pos_pad = jnp.pad(pos, ((0, np_rows - n), (0, 128 - pos.shape[1])))
    wx = W_local[:d]
    wv = jnp.pad(W_local[d:], ((0, 128 - (W_local.shape[0] - d)), (0, 0)))
    src = jnp.pad(edge_index[0], (0, ep - e))
    dst = jnp.pad(edge_index[1], (0, ep - e), constant_values=1 << 30)

    blk_rows = 512
    grid = (np_rows // blk_rows,)
    v, xs_bf = pl.pallas_call(
        _mm_xs_body,
        grid=grid,
        in_specs=[
            pl.BlockSpec((blk_rows, d), lambda i: (i, 0)),
            pl.BlockSpec((blk_rows, 128), lambda i: (i, 0)),
            pl.BlockSpec((d, d), lambda i: (0, 0)),
            pl.BlockSpec((128, d), lambda i: (0, 0)),
        ],
        out_specs=[
            pl.BlockSpec((blk_rows, d), lambda i: (i, 0)),
            pl.BlockSpec((blk_rows, d), lambda i: (i, 0)),
        ],
        out_shape=[
            jax.ShapeDtypeStruct((np_rows, d), jnp.float32),
            jax.ShapeDtypeStruct((np_rows, d), jnp.bfloat16),
        ],
    )(x_pad, pos_pad, wx, wv)

    dw = d // 2
    xs_pack = lax.bitcast_convert_type(
        xs_bf.reshape(np_rows, dw, 2), jnp.int32)

    part_max = 442368 // dw // 128 * 128
    n_parts = -(-np_rows // part_max)
    part_rows = -(-np_rows // (n_parts * 128)) * 128
    xsp_rows = n_parts * part_rows
    xs_pack = jnp.pad(xs_pack, ((0, xsp_rows - np_rows), (0, 0)))

    m = _sc_segmax(np_rows, dw, ep, r, part_rows, n_parts, xsp_rows)(
        xs_pack, dst, src)

    m_bf = lax.bitcast_convert_type(m, jnp.bfloat16).reshape(np_rows, d)

    out = pl.pallas_call(
        _mm_out_body,
        grid=grid,
        in_specs=[
            pl.BlockSpec((blk_rows, d), lambda i: (i, 0)),
            pl.BlockSpec((blk_rows, d), lambda i: (i, 0)),
            pl.BlockSpec((1, d), lambda i: (0, 0)),
            pl.BlockSpec((d, d), lambda i: (0, 0)),
            pl.BlockSpec((1, d), lambda i: (0, 0)),
        ],
        out_specs=pl.BlockSpec((blk_rows, d), lambda i: (i, 0)),
        out_shape=jax.ShapeDtypeStruct((np_rows, d), jnp.float32),
    )(m_bf, v, b_local.reshape(1, d), W_global, b_global.reshape(1, d))

    return out[:n]


# speedup vs baseline: 2.7142x; 1.0001x over previous
import functools

import jax
import jax.numpy as jnp
from jax import lax
from jax.experimental import pallas as pl
from jax.experimental.pallas import tpu as pltpu
from jax.experimental.pallas import tpu_sc as plsc

NW = 32
LANES = 16
C_EDGES = 2000
K_GATHER = 32


def _mm_xs_body(xb, pb, wx, wv, v_out, xsb_out):
    v = jnp.dot(pb[...], wv[...], preferred_element_type=jnp.float32)
    v_out[...] = v
    xs = jnp.dot(xb[...].astype(jnp.bfloat16), wx[...].astype(jnp.bfloat16),
                 preferred_element_type=jnp.float32) + v
    xsb_out[...] = xs.astype(jnp.bfloat16)


def _mm_out_body(mb, vb, blb, wg, bgb, ob):
    h = jnp.maximum(mb[...].astype(jnp.float32) - vb[...] + blb[...], 0.0)
    ob[...] = jnp.dot(h.astype(jnp.bfloat16), wg[...].astype(jnp.bfloat16),
                      preferred_element_type=jnp.float32) + bgb[...]


G_RING = 8


def _sc_segmax(np_rows, dw, ep, r, part_rows, n_parts):
    n_rounds = ep // C_EDGES
    n_grp = C_EDGES // LANES
    n_sub = dw // LANES
    cl = C_EDGES + 3 * LANES
    mesh = plsc.VectorSubcoreMesh(core_axis_name="c", subcore_axis_name="s")

    @functools.partial(
        pl.kernel,
        mesh=mesh,
        compiler_params=pltpu.CompilerParams(needs_layout_passes=False),
        out_type=jax.ShapeDtypeStruct((np_rows, dw), jnp.int32),
        scratch_types=[
            pltpu.VMEM((r + 1, dw), jnp.int32),
            pltpu.VMEM((2 * C_EDGES,), jnp.int32),
            pltpu.VMEM((2 * cl,), jnp.int32),
            pltpu.VMEM((2 * cl,), jnp.int32),
            pltpu.VMEM((G_RING * K_GATHER, dw), jnp.int32),
            pltpu.VMEM_SHARED((part_rows, dw), jnp.int32),
            pltpu.SemaphoreType.DMA,
            pltpu.SemaphoreType.DMA,
            pltpu.SemaphoreType.DMA,
        ],
    )
    def seg_max(xs_hbm, edge_hbm, m_hbm,
                m_loc, ebuf, slist, dloc, rows, xs_sp, esem, gsem, ssem):
        cid = lax.axis_index("c")
        sid = lax.axis_index("s")
        wid = sid * 2 + cid
        lo = wid * r
        iota = lax.iota(jnp.int32, LANES)

        def start_edges(ci):
            p = ci % 2
            base = ci * C_EDGES
            pltpu.async_copy(edge_hbm.at[pl.ds(base, C_EDGES)],
                             ebuf.at[pl.ds(p * C_EDGES, C_EDGES)], esem)

        def wait_edges(p):
            pltpu.make_async_copy(
                edge_hbm.at[pl.ds(0, C_EDGES)],
                ebuf.at[pl.ds(p * C_EDGES, C_EDGES)], esem).wait()

        def issue(par, blk_loc, slot):
            pltpu.async_copy(
                xs_sp.at[slist.at[pl.ds(par * cl + blk_loc * K_GATHER,
                                        K_GATHER)]],
                rows.at[pl.ds(slot * K_GATHER, K_GATHER)], gsem)

        def wait_gather(slot):
            pltpu.make_async_copy(
                xs_hbm.at[pl.ds(0, K_GATHER)],
                rows.at[pl.ds(slot * K_GATHER, K_GATHER)], gsem).wait()

        pltpu.sync_copy(xs_hbm.at[pl.ds(lo, r)], m_loc.at[pl.ds(0, r)])

        def round_body(base_s, ci, carry):
            pq, iq, b_prev, nb_prev = carry
            p = ci % 2
            wait_edges(p)

            @pl.when(ci + 1 < n_rounds)
            def _():
                start_edges(ci + 1)

            def grp(g, n):
                p16 = ebuf[pl.ds(p * C_EDGES + g * LANES, LANES)]
                d16 = lax.shift_right_logical(p16, 16)
                sl16 = (p16 & 0xFFFF) - base_s
                msk = ((d16 - lo).astype(jnp.uint32) < jnp.uint32(r)) & (
                    sl16.astype(jnp.uint32) < jnp.uint32(part_rows))
                plsc.store_compressed(
                    slist.at[pl.ds(p * cl + n, LANES)], sl16, mask=msk)
                plsc.store_compressed(
                    dloc.at[pl.ds(p * cl + n, LANES)], d16 - lo, mask=msk)
                return n + plsc.all_reduce_population_count(msk)[0]

            n = lax.fori_loop(0, n_grp, grp, jnp.int32(0), unroll=2)
            slist[pl.ds(p * cl + n, LANES)] = jnp.zeros((LANES,), jnp.int32)
            slist[pl.ds(p * cl + n + LANES, LANES)] = jnp.zeros(
                (LANES,), jnp.int32)
            dloc[pl.ds(p * cl + n, LANES)] = jnp.full((LANES,), r, jnp.int32)
            dloc[pl.ds(p * cl + n + LANES, LANES)] = jnp.full(
                (LANES,), r, jnp.int32)
            nb_cur = (n + K_GATHER - 1) // K_GATHER
            b_cur = b_prev + nb_prev
            frontier = b_cur + nb_cur

            def issue_global(q):
                is_cur = q >= b_cur
                par = jnp.where(is_cur, p, 1 - p)
                loc = q - jnp.where(is_cur, b_cur, b_prev)
                issue(par, loc, q % G_RING)

            def seed(k, iq2):
                can = (iq2 < frontier) & (iq2 - pq < G_RING)

                @pl.when(can)
                def _():
                    issue_global(iq2)

                return iq2 + can.astype(jnp.int32)

            iq = lax.fori_loop(0, G_RING, seed, iq)

            def pbody(b, carry2):
                pq2, iq2 = carry2
                g = b_prev + b

                @pl.when(iq2 == g)
                def _():
                    issue_global(g)

                iq2 = jnp.maximum(iq2, g + 1)
                slot = g % G_RING
                wait_gather(slot)

                def grp16(h, carry3):
                    dl16 = dloc[pl.ds((1 - p) * cl + b * K_GATHER + h * LANES,
                                      LANES)]

                    def edge(i, carry4):
                        dlv = dl16.at[jnp.full((LANES,), i, jnp.int32)].get(
                            mode="promise_in_bounds")
                        gs, rs = [], []
                        for j in range(n_sub):
                            gs.append(plsc.load_gather(
                                m_loc, [dlv, iota + (j * LANES)]))
                            rs.append(rows[slot * K_GATHER + h * LANES + i,
                                           pl.ds(j * LANES, LANES)])
                        mx = [
                            plsc.bitcast(
                                jnp.maximum(plsc.bitcast(a, jnp.bfloat16),
                                            plsc.bitcast(c, jnp.bfloat16)),
                                jnp.int32)
                            for a, c in zip(gs, rs)
                        ]
                        for j in range(n_sub):
                            plsc.store_scatter(
                                m_loc, [dlv, iota + (j * LANES)], mx[j])
                        return carry4

                    lax.fori_loop(0, LANES, edge, jnp.int32(0), unroll=2)
                    return carry3

                lax.fori_loop(0, K_GATHER // LANES, grp16, jnp.int32(0))
                pq2 = g + 1
                can = (iq2 < frontier) & (iq2 - pq2 < G_RING)

                @pl.when(can)
                def _():
                    issue_global(iq2)

                return pq2, iq2 + can.astype(jnp.int32)

            pq, iq = lax.fori_loop(0, nb_prev, pbody, (pq, iq))
            return pq, iq, b_cur, nb_cur

        z = jnp.int32(0)
        s_rows = part_rows // 16
        for t in range(n_parts):
            base_t = min(t * part_rows, np_rows - part_rows)
            pltpu.async_copy(
                xs_hbm.at[pl.ds(base_t + sid * s_rows, s_rows)],
                xs_sp.at[pl.ds(sid * s_rows, s_rows)], ssem)
            pltpu.make_async_copy(
                xs_hbm.at[pl.ds(0, s_rows)],
                xs_sp.at[pl.ds(sid * s_rows, s_rows)], ssem).wait()
            plsc.subcore_barrier()
            start_edges(0)
            lax.fori_loop(0, n_rounds,
                          functools.partial(round_body, base_t),
                          (z, z, z, z))
            plsc.subcore_barrier()
        pltpu.sync_copy(m_loc.at[pl.ds(0, r)], m_hbm.at[pl.ds(lo, r)])

    return seg_max


def kernel(x, pos, edge_index, W_local, b_local, W_global, b_global):
    n, d = x.shape
    e = edge_index.shape[1]
    r = -(-n // (NW * 8)) * 8
    np_rows = NW * r
    ep = (-(-e // C_EDGES) + 1) * C_EDGES

    x_pad = jnp.pad(x, ((0, np_rows - n), (0, 0)))
    pos_pad = jnp.pad(pos, ((0, np_rows - n), (0, 128 - pos.shape[1])))
    wx = W_local[:d]
    wv = jnp.pad(W_local[d:], ((0, 128 - (W_local.shape[0] - d)), (0, 0)))
    src = jnp.pad(edge_index[0], (0, ep - e))
    dst = jnp.pad(edge_index[1], (0, ep - e), constant_values=0xFFFF)
    edges = jnp.left_shift(dst, 16) | src

    blk_rows = 512
    grid = (np_rows // blk_rows,)
    v, xs_bf = pl.pallas_call(
        _mm_xs_body,
        grid=grid,
        in_specs=[
            pl.BlockSpec((blk_rows, d), lambda i: (i, 0)),
            pl.BlockSpec((blk_rows, 128), lambda i: (i, 0)),
            pl.BlockSpec((d, d), lambda i: (0, 0)),
            pl.BlockSpec((128, d), lambda i: (0, 0)),
        ],
        out_specs=[
            pl.BlockSpec((blk_rows, d), lambda i: (i, 0)),
            pl.BlockSpec((blk_rows, d), lambda i: (i, 0)),
        ],
        out_shape=[
            jax.ShapeDtypeStruct((np_rows, d), jnp.float32),
            jax.ShapeDtypeStruct((np_rows, d), jnp.bfloat16),
        ],
    )(x_pad, pos_pad, wx, wv)

    dw = d // 2
    xs_pack = lax.bitcast_convert_type(
        xs_bf.reshape(np_rows, dw, 2), jnp.int32)

    part_max = 442368 // dw // 128 * 128
    n_parts = -(-np_rows // part_max)
    part_rows = -(-np_rows // (n_parts * 128)) * 128

    m = _sc_segmax(np_rows, dw, ep, r, part_rows, n_parts)(xs_pack, edges)

    m_bf = lax.bitcast_convert_type(m, jnp.bfloat16).reshape(np_rows, d)

    out = pl.pallas_call(
        _mm_out_body,
        grid=grid,
        in_specs=[
            pl.BlockSpec((blk_rows, d), lambda i: (i, 0)),
            pl.BlockSpec((blk_rows, d), lambda i: (i, 0)),
            pl.BlockSpec((1, d), lambda i: (0, 0)),
            pl.BlockSpec((d, d), lambda i: (0, 0)),
            pl.BlockSpec((1, d), lambda i: (0, 0)),
        ],
        out_specs=pl.BlockSpec((blk_rows, d), lambda i: (i, 0)),
        out_shape=jax.ShapeDtypeStruct((np_rows, d), jnp.float32),
    )(m_bf, v, b_local.reshape(1, d), W_global, b_global.reshape(1, d))

    return out[:n]
